# baseline (speedup 1.0000x reference)
import math

import jax
import jax.numpy as jnp
from jax.experimental import pallas as pl

_N = 2048
_IN_DIM = 768
_DIM = 512
_DEPTH = 2
_HEADS = 8
_KB = 10
_MLP = 2048
_NC = 2
_DH = _DIM // _HEADS

_NP1 = _N + 1
_TQ = 128
_NPAD = ((_NP1 + _TQ - 1) // _TQ) * _TQ
_NQT = _NPAD // _TQ
_NXT = _N // _TQ

_BF = jnp.bfloat16
_F32 = jnp.float32


def _lnorm(x, g, b, eps=1e-5):
    m = jnp.mean(x, axis=-1, keepdims=True)
    v = jnp.mean((x - m) ** 2, axis=-1, keepdims=True)
    return (x - m) / jnp.sqrt(v + eps) * g + b


def _embed_kernel(x_ref, w_ref, b_ref, o_ref):
    acc = jnp.dot(x_ref[...], w_ref[...], preferred_element_type=_F32)
    o_ref[...] = jax.nn.gelu(acc + b_ref[...])


def _qkv_kernel(h_ref, g_ref, b_ref, wq_ref, bq_ref, wk_ref, bk_ref,
                wv_ref, bv_ref, q_ref, k_ref, v_ref):
    scale = 1.0 / math.sqrt(_DH)
    xn = _lnorm(h_ref[...], g_ref[...], b_ref[...]).astype(_BF)
    q_ref[...] = ((jnp.dot(xn, wq_ref[...], preferred_element_type=_F32)
                   + bq_ref[...]) * scale).astype(_BF)
    k_ref[...] = (jnp.dot(xn, wk_ref[...], preferred_element_type=_F32)
                  + bk_ref[...]).astype(_BF)
    vf = (jnp.dot(xn, wv_ref[...], preferred_element_type=_F32)
          + bv_ref[...]).astype(_BF)
    ones = jnp.ones((vf.shape[0], 1), _BF)
    zeros = jnp.zeros((vf.shape[0], 128 - _DH - 1), _BF)
    slabs = []
    for hh in range(_HEADS):
        slabs.append(vf[:, hh * _DH:(hh + 1) * _DH])
        slabs.append(ones)
        slabs.append(zeros)
    v_ref[...] = jnp.concatenate(slabs, axis=1)


def _layer_kernel(q_ref, k_ref, v_ref, h_ref, cq_ref, ck_ref, drel_ref,
                  wo_ref, bo_ref, g2_ref, b2_ref, w1_ref, b1_ref,
                  w2_ref, b2b_ref, out_ref):
    qi = pl.program_id(0)

    qx = cq_ref[0, :, 0:1]
    qy = cq_ref[0, :, 1:2]
    kx = ck_ref[0:1, :]
    ky = ck_ref[1:2, :]
    dx = qx - kx
    dy = qy - ky
    d2 = (dx * dx + dy * dy).astype(_BF)
    masks = [d2 >= _BF((t / _KB) ** 2 - 1e-12) for t in range(1, _KB)]

    rows = qi * _TQ + jax.lax.broadcasted_iota(jnp.int32, (_TQ, 1), 0)
    cols = jax.lax.broadcasted_iota(jnp.int32, (1, _NPAD), 1)
    row0b = (rows == 0).astype(_BF)
    col0b = (cols == 0).astype(_BF)
    mask0 = (row0b + col0b) > _BF(0.5)
    kmask = jnp.where(cols > _N, -1e30, 0.0).astype(_BF)

    qf = q_ref[...]
    kf = k_ref[...]
    vf = v_ref[...]
    ss = []
    for hh in range(_HEADS):
        sl = slice(hh * _DH, (hh + 1) * _DH)
        ss.append(jax.lax.dot_general(qf[:, sl], kf[:, sl],
                                      (((1,), (1,)), ((), ())),
                                      preferred_element_type=_F32).astype(_BF))
    biases = [jnp.broadcast_to(drel_ref[hh:hh + 1, 0:1].astype(_BF),
                               ss[0].shape) for hh in range(_HEADS)]
    for t in range(1, _KB):
        m = masks[t - 1]
        for hh in range(_HEADS):
            rt = drel_ref[hh:hh + 1, t:t + 1].astype(_BF)
            biases[hh] = jnp.where(m, rt, biases[hh])
    ohs = []
    for hh in range(_HEADS):
        bias = jnp.where(mask0, _BF(0.0), biases[hh])
        e = jnp.exp(ss[hh] + bias + kmask)
        ohx = jnp.dot(e, vf[:, hh * 128:(hh + 1) * 128],
                      preferred_element_type=_F32)
        ohs.append(ohx[:, :_DH] * (1.0 / ohx[:, _DH:_DH + 1]))
    oc = jnp.concatenate(ohs, axis=1).astype(_BF)

    h2 = h_ref[...] + jnp.dot(oc, wo_ref[...],
                              preferred_element_type=_F32) + bo_ref[...]
    xn = _lnorm(h2, g2_ref[...], b2_ref[...]).astype(_BF)
    hh1 = jax.nn.gelu(jnp.dot(xn, w1_ref[...],
                              preferred_element_type=_F32) + b1_ref[...])
    ff = jnp.dot(hh1.astype(_BF), w2_ref[...],
                 preferred_element_type=_F32) + b2b_ref[...]
    out_ref[...] = h2 + ff


def _final_kernel(h_ref, g_ref, b_ref, w_ref, bh_ref, o_ref):
    x0 = h_ref[0:1, :]
    xn = _lnorm(x0, g_ref[...], b_ref[...])
    o_ref[...] = jnp.dot(xn, w_ref[...], preferred_element_type=_F32) + bh_ref[...]


def kernel(x, coord, lens, cls_token, fc_w, fc_b, ln1_g, ln1_b, wq, bq,
           wk, bk, wv, bv, wo, bo, rel_bias, ln2_g, ln2_b, w1, b1, w2, b2,
           lnf_g, lnf_b, head_w, head_b):
    x2 = x[0]

    emb = pl.pallas_call(
        _embed_kernel,
        grid=(_NXT,),
        in_specs=[
            pl.BlockSpec((_TQ, _IN_DIM), lambda i: (i, 0)),
            pl.BlockSpec((_IN_DIM, _DIM), lambda i: (0, 0)),
            pl.BlockSpec((1, _DIM), lambda i: (0, 0)),
        ],
        out_specs=pl.BlockSpec((_TQ, _DIM), lambda i: (i, 0)),
        out_shape=jax.ShapeDtypeStruct((_N, _DIM), _F32),
    )(x2, fc_w, fc_b.reshape(1, _DIM))

    h = jnp.concatenate([
        cls_token.reshape(1, _DIM),
        emb,
        jnp.zeros((_NPAD - _NP1, _DIM), _F32),
    ], axis=0)

    pc = jnp.concatenate([
        jnp.zeros((1, 2), _F32),
        coord[0],
        jnp.zeros((_NPAD - _NP1, 2), _F32),
    ], axis=0)
    cq = jnp.pad(pc, ((0, 0), (0, 6))).reshape(_NQT, _TQ, 8)
    ck = jnp.pad(pc.T, ((0, 6), (0, 0)))

    for l in range(_DEPTH):
        q, k, v = pl.pallas_call(
            _qkv_kernel,
            grid=(_NQT,),
            in_specs=[
                pl.BlockSpec((_TQ, _DIM), lambda i: (i, 0)),
                pl.BlockSpec((1, _DIM), lambda i: (0, 0)),
                pl.BlockSpec((1, _DIM), lambda i: (0, 0)),
                pl.BlockSpec((_DIM, _DIM), lambda i: (0, 0)),
                pl.BlockSpec((1, _DIM), lambda i: (0, 0)),
                pl.BlockSpec((_DIM, _DIM), lambda i: (0, 0)),
                pl.BlockSpec((1, _DIM), lambda i: (0, 0)),
                pl.BlockSpec((_DIM, _DIM), lambda i: (0, 0)),
                pl.BlockSpec((1, _DIM), lambda i: (0, 0)),
            ],
            out_specs=[
                pl.BlockSpec((_TQ, _DIM), lambda i: (i, 0)),
                pl.BlockSpec((_TQ, _DIM), lambda i: (i, 0)),
                pl.BlockSpec((_TQ, _HEADS * 128), lambda i: (i, 0)),
            ],
            out_shape=[
                jax.ShapeDtypeStruct((_NPAD, _DIM), _BF),
                jax.ShapeDtypeStruct((_NPAD, _DIM), _BF),
                jax.ShapeDtypeStruct((_NPAD, _HEADS * 128), _BF),
            ],
        )(h, ln1_g[l].reshape(1, _DIM), ln1_b[l].reshape(1, _DIM),
          wq[l].astype(_BF), bq[l].reshape(1, _DIM),
          wk[l].astype(_BF), bk[l].reshape(1, _DIM),
          wv[l].astype(_BF), bv[l].reshape(1, _DIM))

        drel = rel_bias[l].T

        h = pl.pallas_call(
            _layer_kernel,
            grid=(_NQT,),
            in_specs=[
                pl.BlockSpec((_TQ, _DIM), lambda i: (i, 0)),
                pl.BlockSpec((_NPAD, _DIM), lambda i: (0, 0)),
                pl.BlockSpec((_NPAD, _HEADS * 128), lambda i: (0, 0)),
                pl.BlockSpec((_TQ, _DIM), lambda i: (i, 0)),
                pl.BlockSpec((1, _TQ, 8), lambda i: (i, 0, 0)),
                pl.BlockSpec((8, _NPAD), lambda i: (0, 0)),
                pl.BlockSpec((_HEADS, _KB), lambda i: (0, 0)),
                pl.BlockSpec((_DIM, _DIM), lambda i: (0, 0)),
                pl.BlockSpec((1, _DIM), lambda i: (0, 0)),
                pl.BlockSpec((1, _DIM), lambda i: (0, 0)),
                pl.BlockSpec((1, _DIM), lambda i: (0, 0)),
                pl.BlockSpec((_DIM, _MLP), lambda i: (0, 0)),
                pl.BlockSpec((1, _MLP), lambda i: (0, 0)),
                pl.BlockSpec((_MLP, _DIM), lambda i: (0, 0)),
                pl.BlockSpec((1, _DIM), lambda i: (0, 0)),
            ],
            out_specs=pl.BlockSpec((_TQ, _DIM), lambda i: (i, 0)),
            out_shape=jax.ShapeDtypeStruct((_NPAD, _DIM), _F32),
        )(q, k, v, h, cq, ck, drel,
          wo[l].astype(_BF), bo[l].reshape(1, _DIM),
          ln2_g[l].reshape(1, _DIM), ln2_b[l].reshape(1, _DIM),
          w1[l].astype(_BF), b1[l].reshape(1, _MLP),
          w2[l].astype(_BF), b2[l].reshape(1, _DIM))

    hw = jnp.pad(head_w, ((0, 0), (0, 128 - _NC)))
    hb = jnp.pad(head_b, (0, 128 - _NC)).reshape(1, 128)
    out = pl.pallas_call(
        _final_kernel,
        grid=(1,),
        in_specs=[
            pl.BlockSpec((8, _DIM), lambda i: (0, 0)),
            pl.BlockSpec((1, _DIM), lambda i: (0, 0)),
            pl.BlockSpec((1, _DIM), lambda i: (0, 0)),
            pl.BlockSpec((_DIM, 128), lambda i: (0, 0)),
            pl.BlockSpec((1, 128), lambda i: (0, 0)),
        ],
        out_specs=pl.BlockSpec((1, 128), lambda i: (0, 0)),
        out_shape=jax.ShapeDtypeStruct((1, 128), _F32),
    )(h, lnf_g.reshape(1, _DIM), lnf_b.reshape(1, _DIM), hw, hb)

    return out[:, :_NC]


# speedup vs baseline: 1.0653x; 1.0653x over previous
import math

import jax
import jax.numpy as jnp
from jax.experimental import pallas as pl

_N = 2048
_IN_DIM = 768
_DIM = 512
_HEADS = 8
_KB = 10
_MLP = 2048
_NC = 2
_DH = _DIM // _HEADS

_NP1 = _N + 1
_TQ = 128
_NPAD = ((_NP1 + _TQ - 1) // _TQ) * _TQ
_NQT = _NPAD // _TQ
_VW = _HEADS * 128

_BF = jnp.bfloat16
_F32 = jnp.float32


def _lnorm(x, g, b, eps=1e-5):
    m = jnp.mean(x, axis=-1, keepdims=True)
    v = jnp.mean((x - m) ** 2, axis=-1, keepdims=True)
    return (x - m) / jnp.sqrt(v + eps) * g + b


def _qkv(xn, wq_ref, bq_ref, wk_ref, bk_ref, wv_ref, bv_ref):
    scale = 1.0 / math.sqrt(_DH)
    q = ((jnp.dot(xn, wq_ref[...], preferred_element_type=_F32)
          + bq_ref[...]) * scale).astype(_BF)
    k = (jnp.dot(xn, wk_ref[...], preferred_element_type=_F32)
         + bk_ref[...]).astype(_BF)
    vf = (jnp.dot(xn, wv_ref[...], preferred_element_type=_F32)
          + bv_ref[...]).astype(_BF)
    ones = jnp.ones((vf.shape[0], 1), _BF)
    zeros = jnp.zeros((vf.shape[0], 128 - _DH - 1), _BF)
    slabs = []
    for hh in range(_HEADS):
        slabs.append(vf[:, hh * _DH:(hh + 1) * _DH])
        slabs.append(ones)
        slabs.append(zeros)
    return q, k, jnp.concatenate(slabs, axis=1)


def _entry_kernel(xp_ref, cls_ref, fcw_ref, fcb_ref, g_ref, b_ref,
                  wq_ref, bq_ref, wk_ref, bk_ref, wv_ref, bv_ref,
                  h_ref, q_ref, k_ref, v_ref):
    qi = pl.program_id(0)
    emb = jax.nn.gelu(jnp.dot(xp_ref[...], fcw_ref[...],
                              preferred_element_type=_F32) + fcb_ref[...])
    rows = qi * _TQ + jax.lax.broadcasted_iota(jnp.int32, (_TQ, 1), 0)
    h = jnp.where(rows == 0, cls_ref[...], emb)
    h_ref[...] = h
    xn = _lnorm(h, g_ref[...], b_ref[...]).astype(_BF)
    q, k, v = _qkv(xn, wq_ref, bq_ref, wk_ref, bk_ref, wv_ref, bv_ref)
    q_ref[...] = q
    k_ref[...] = k
    v_ref[...] = v


def _attn_ffn(qi, q_ref, k_ref, v_ref, h_ref, cq_ref, ck_ref, drel_ref,
              wo_ref, bo_ref, g2_ref, b2_ref, w1_ref, b1_ref,
              w2_ref, b2b_ref):
    qx = cq_ref[0, :, 0:1]
    qy = cq_ref[0, :, 1:2]
    kx = ck_ref[0:1, :]
    ky = ck_ref[1:2, :]
    dx = qx - kx
    dy = qy - ky
    d2 = (dx * dx + dy * dy).astype(_BF)
    masks = [d2 >= _BF((t / _KB) ** 2 - 1e-12) for t in range(1, _KB)]

    rows = qi * _TQ + jax.lax.broadcasted_iota(jnp.int32, (_TQ, 1), 0)
    cols = jax.lax.broadcasted_iota(jnp.int32, (1, _NPAD), 1)
    row0b = (rows == 0).astype(_BF)
    col0b = (cols == 0).astype(_BF)
    mask0 = (row0b + col0b) > _BF(0.5)
    kmask = jnp.where(cols > _N, -1e30, 0.0).astype(_BF)

    qf = q_ref[...]
    kf = k_ref[...]
    vf = v_ref[...]
    ohs = []
    for hh in range(_HEADS):
        sl = slice(hh * _DH, (hh + 1) * _DH)
        s = jax.lax.dot_general(qf[:, sl], kf[:, sl], (((1,), (1,)), ((), ())),
                                preferred_element_type=_F32).astype(_BF)
        r0 = drel_ref[hh:hh + 1, 0:1].astype(_BF)
        bias = jnp.broadcast_to(r0, s.shape)
        for t in range(1, _KB):
            rt = drel_ref[hh:hh + 1, t:t + 1].astype(_BF)
            bias = jnp.where(masks[t - 1], rt, bias)
        bias = jnp.where(mask0, _BF(0.0), bias)
        e = jnp.exp(s + bias + kmask)
        ohx = jnp.dot(e, vf[:, hh * 128:(hh + 1) * 128],
                      preferred_element_type=_F32)
        ohs.append(ohx[:, :_DH] * (1.0 / ohx[:, _DH:_DH + 1]))
    oc = jnp.concatenate(ohs, axis=1).astype(_BF)

    h2 = h_ref[...] + jnp.dot(oc, wo_ref[...],
                              preferred_element_type=_F32) + bo_ref[...]
    xn = _lnorm(h2, g2_ref[...], b2_ref[...]).astype(_BF)
    hh1 = jax.nn.gelu(jnp.dot(xn, w1_ref[...],
                              preferred_element_type=_F32) + b1_ref[...])
    ff = jnp.dot(hh1.astype(_BF), w2_ref[...],
                 preferred_element_type=_F32) + b2b_ref[...]
    return h2 + ff


def _mid_kernel(q_ref, k_ref, v_ref, h_ref, cq_ref, ck_ref, drel_ref,
                wo_ref, bo_ref, g2_ref, b2_ref, w1_ref, b1_ref,
                w2_ref, b2b_ref, g1n_ref, b1n_ref,
                wqn_ref, bqn_ref, wkn_ref, bkn_ref, wvn_ref, bvn_ref,
                ho_ref, qo_ref, ko_ref, vo_ref):
    qi = pl.program_id(0)
    hn = _attn_ffn(qi, q_ref, k_ref, v_ref, h_ref, cq_ref, ck_ref, drel_ref,
                   wo_ref, bo_ref, g2_ref, b2_ref, w1_ref, b1_ref,
                   w2_ref, b2b_ref)
    ho_ref[...] = hn
    xn = _lnorm(hn, g1n_ref[...], b1n_ref[...]).astype(_BF)
    q, k, v = _qkv(xn, wqn_ref, bqn_ref, wkn_ref, bkn_ref, wvn_ref, bvn_ref)
    qo_ref[...] = q
    ko_ref[...] = k
    vo_ref[...] = v


def _last_kernel(q_ref, k_ref, v_ref, h_ref, cq_ref, ck_ref, drel_ref,
                 wo_ref, bo_ref, g2_ref, b2_ref, w1_ref, b1_ref,
                 w2_ref, b2b_ref, gf_ref, bf_ref, hw_ref, hb_ref, o_ref):
    qi = pl.program_id(0)
    hn = _attn_ffn(qi, q_ref, k_ref, v_ref, h_ref, cq_ref, ck_ref, drel_ref,
                   wo_ref, bo_ref, g2_ref, b2_ref, w1_ref, b1_ref,
                   w2_ref, b2b_ref)

    @pl.when(qi == 0)
    def _():
        x0 = hn[0:1, :]
        xn = _lnorm(x0, gf_ref[...], bf_ref[...])
        o_ref[...] = (jnp.dot(xn, hw_ref[...], preferred_element_type=_F32)
                      + hb_ref[...])


def kernel(x, coord, lens, cls_token, fc_w, fc_b, ln1_g, ln1_b, wq, bq,
           wk, bk, wv, bv, wo, bo, rel_bias, ln2_g, ln2_b, w1, b1, w2, b2,
           lnf_g, lnf_b, head_w, head_b):
    xp = jnp.concatenate([
        jnp.zeros((1, _IN_DIM), _F32),
        x[0],
        jnp.zeros((_NPAD - _NP1, _IN_DIM), _F32),
    ], axis=0).astype(_BF)

    pc = jnp.concatenate([
        jnp.zeros((1, 2), _F32),
        coord[0],
        jnp.zeros((_NPAD - _NP1, 2), _F32),
    ], axis=0)
    cq = jnp.pad(pc, ((0, 0), (0, 6))).reshape(_NQT, _TQ, 8)
    ck = jnp.pad(pc.T, ((0, 6), (0, 0)))

    row_spec = pl.BlockSpec((_TQ, _DIM), lambda i: (i, 0))
    vec_spec = pl.BlockSpec((1, _DIM), lambda i: (0, 0))
    full_spec = lambda a, b: pl.BlockSpec((a, b), lambda i: (0, 0))
    slab_spec = pl.BlockSpec((_TQ, _VW), lambda i: (i, 0))

    h, q, k, v = pl.pallas_call(
        _entry_kernel,
        grid=(_NQT,),
        in_specs=[
            pl.BlockSpec((_TQ, _IN_DIM), lambda i: (i, 0)),
            vec_spec,
            full_spec(_IN_DIM, _DIM),
            vec_spec, vec_spec, vec_spec,
            full_spec(_DIM, _DIM), vec_spec,
            full_spec(_DIM, _DIM), vec_spec,
            full_spec(_DIM, _DIM), vec_spec,
        ],
        out_specs=[row_spec, row_spec, row_spec, slab_spec],
        out_shape=[
            jax.ShapeDtypeStruct((_NPAD, _DIM), _F32),
            jax.ShapeDtypeStruct((_NPAD, _DIM), _BF),
            jax.ShapeDtypeStruct((_NPAD, _DIM), _BF),
            jax.ShapeDtypeStruct((_NPAD, _VW), _BF),
        ],
    )(xp, cls_token.reshape(1, _DIM), fc_w.astype(_BF),
      fc_b.reshape(1, _DIM), ln1_g[0].reshape(1, _DIM),
      ln1_b[0].reshape(1, _DIM),
      wq[0].astype(_BF), bq[0].reshape(1, _DIM),
      wk[0].astype(_BF), bk[0].reshape(1, _DIM),
      wv[0].astype(_BF), bv[0].reshape(1, _DIM))

    layer_specs = [
        row_spec,
        pl.BlockSpec((_NPAD, _DIM), lambda i: (0, 0)),
        pl.BlockSpec((_NPAD, _VW), lambda i: (0, 0)),
        row_spec,
        pl.BlockSpec((1, _TQ, 8), lambda i: (i, 0, 0)),
        full_spec(8, _NPAD),
        full_spec(_HEADS, _KB),
        full_spec(_DIM, _DIM), vec_spec,
        vec_spec, vec_spec,
        full_spec(_DIM, _MLP), pl.BlockSpec((1, _MLP), lambda i: (0, 0)),
        full_spec(_MLP, _DIM), vec_spec,
    ]

    h, q, k, v = pl.pallas_call(
        _mid_kernel,
        grid=(_NQT,),
        in_specs=layer_specs + [
            vec_spec, vec_spec,
            full_spec(_DIM, _DIM), vec_spec,
            full_spec(_DIM, _DIM), vec_spec,
            full_spec(_DIM, _DIM), vec_spec,
        ],
        out_specs=[row_spec, row_spec, row_spec, slab_spec],
        out_shape=[
            jax.ShapeDtypeStruct((_NPAD, _DIM), _F32),
            jax.ShapeDtypeStruct((_NPAD, _DIM), _BF),
            jax.ShapeDtypeStruct((_NPAD, _DIM), _BF),
            jax.ShapeDtypeStruct((_NPAD, _VW), _BF),
        ],
    )(q, k, v, h, cq, ck, rel_bias[0].T,
      wo[0].astype(_BF), bo[0].reshape(1, _DIM),
      ln2_g[0].reshape(1, _DIM), ln2_b[0].reshape(1, _DIM),
      w1[0].astype(_BF), b1[0].reshape(1, _MLP),
      w2[0].astype(_BF), b2[0].reshape(1, _DIM),
      ln1_g[1].reshape(1, _DIM), ln1_b[1].reshape(1, _DIM),
      wq[1].astype(_BF), bq[1].reshape(1, _DIM),
      wk[1].astype(_BF), bk[1].reshape(1, _DIM),
      wv[1].astype(_BF), bv[1].reshape(1, _DIM))

    hw = jnp.pad(head_w, ((0, 0), (0, 128 - _NC)))
    hb = jnp.pad(head_b, (0, 128 - _NC)).reshape(1, 128)
    out = pl.pallas_call(
        _last_kernel,
        grid=(_NQT,),
        in_specs=layer_specs + [
            vec_spec, vec_spec,
            full_spec(_DIM, 128),
            pl.BlockSpec((1, 128), lambda i: (0, 0)),
        ],
        out_specs=pl.BlockSpec((1, 128), lambda i: (0, 0)),
        out_shape=jax.ShapeDtypeStruct((1, 128), _F32),
    )(q, k, v, h, cq, ck, rel_bias[1].T,
      wo[1].astype(_BF), bo[1].reshape(1, _DIM),
      ln2_g[1].reshape(1, _DIM), ln2_b[1].reshape(1, _DIM),
      w1[1].astype(_BF), b1[1].reshape(1, _MLP),
      w2[1].astype(_BF), b2[1].reshape(1, _DIM),
      lnf_g.reshape(1, _DIM), lnf_b.reshape(1, _DIM), hw, hb)

    return out[:, :_NC]


# speedup vs baseline: 1.2934x; 1.2140x over previous
import math

import jax
import jax.numpy as jnp
from jax.experimental import pallas as pl

_N = 2048
_IN_DIM = 768
_DIM = 512
_HEADS = 8
_KB = 10
_MLP = 2048
_NC = 2
_DH = _DIM // _HEADS

_NP1 = _N + 1
_NPAD = 2176
_TQ = 272
_NQT = _NPAD // _TQ
_VW = _HEADS * 128

_BF = jnp.bfloat16
_F32 = jnp.float32


def _lnorm(x, g, b, eps=1e-5):
    m = jnp.mean(x, axis=-1, keepdims=True)
    v = jnp.mean((x - m) ** 2, axis=-1, keepdims=True)
    return (x - m) / jnp.sqrt(v + eps) * g + b


def _qkv(xn, wq_ref, bq_ref, wk_ref, bk_ref, wv_ref, bv_ref):
    scale = 1.0 / math.sqrt(_DH)
    q = ((jnp.dot(xn, wq_ref[...], preferred_element_type=_F32)
          + bq_ref[...]) * scale).astype(_BF)
    k = (jnp.dot(xn, wk_ref[...], preferred_element_type=_F32)
         + bk_ref[...]).astype(_BF)
    vf = (jnp.dot(xn, wv_ref[...], preferred_element_type=_F32)
          + bv_ref[...]).astype(_BF)
    ones = jnp.ones((vf.shape[0], 1), _BF)
    zeros = jnp.zeros((vf.shape[0], 128 - _DH - 1), _BF)
    slabs = []
    for hh in range(_HEADS):
        slabs.append(vf[:, hh * _DH:(hh + 1) * _DH])
        slabs.append(ones)
        slabs.append(zeros)
    return q, k, jnp.concatenate(slabs, axis=1)


def _entry_kernel(xp_ref, cls_ref, fcw_ref, fcb_ref, g_ref, b_ref,
                  wq_ref, bq_ref, wk_ref, bk_ref, wv_ref, bv_ref,
                  h_ref, q_ref, k_ref, v_ref):
    qi = pl.program_id(0)
    emb = jax.nn.gelu(jnp.dot(xp_ref[...], fcw_ref[...],
                              preferred_element_type=_F32) + fcb_ref[...])
    rows = qi * _TQ + jax.lax.broadcasted_iota(jnp.int32, (_TQ, 1), 0)
    h = jnp.where(rows == 0, cls_ref[...], emb)
    h_ref[...] = h
    xn = _lnorm(h, g_ref[...], b_ref[...]).astype(_BF)
    q, k, v = _qkv(xn, wq_ref, bq_ref, wk_ref, bk_ref, wv_ref, bv_ref)
    q_ref[...] = q
    k_ref[...] = k
    v_ref[...] = v


def _attn_ffn(qi, q_ref, k_ref, v_ref, h_ref, cq_ref, ck_ref, drel_ref,
              wo_ref, bo_ref, g2_ref, b2_ref, w1_ref, b1_ref,
              w2_ref, b2b_ref):
    qx = cq_ref[0, :, 0:1]
    qy = cq_ref[0, :, 1:2]
    kx = ck_ref[0:1, :]
    ky = ck_ref[1:2, :]
    dx = qx - kx
    dy = qy - ky
    d2 = (dx * dx + dy * dy).astype(_BF)
    masks = [d2 >= _BF((t / _KB) ** 2 - 1e-12) for t in range(1, _KB)]

    rows = qi * _TQ + jax.lax.broadcasted_iota(jnp.int32, (_TQ, 1), 0)
    cols = jax.lax.broadcasted_iota(jnp.int32, (1, _NPAD), 1)
    row0b = (rows == 0).astype(_BF)
    col0b = (cols == 0).astype(_BF)
    mask0 = (row0b + col0b) > _BF(0.5)
    kmask = jnp.where(cols > _N, -1e30, 0.0).astype(_BF)

    qf = q_ref[...]
    kf = k_ref[...]
    vf = v_ref[...]
    ohs = []
    for hh in range(_HEADS):
        sl = slice(hh * _DH, (hh + 1) * _DH)
        s = jax.lax.dot_general(qf[:, sl], kf[:, sl], (((1,), (1,)), ((), ())),
                                preferred_element_type=_F32).astype(_BF)
        r0 = drel_ref[hh:hh + 1, 0:1].astype(_BF)
        r1 = drel_ref[hh:hh + 1, 1:2].astype(_BF)
        bias = jnp.where(masks[0], r1, r0)
        for t in range(2, _KB):
            rt = drel_ref[hh:hh + 1, t:t + 1].astype(_BF)
            bias = jnp.where(masks[t - 1], rt, bias)
        bias = jnp.where(mask0, _BF(0.0), bias)
        e = jnp.exp(s + bias + kmask)
        ohx = jnp.dot(e, vf[:, hh * 128:(hh + 1) * 128],
                      preferred_element_type=_F32)
        ohs.append(ohx[:, :_DH] * (1.0 / ohx[:, _DH:_DH + 1]))
    oc = jnp.concatenate(ohs, axis=1).astype(_BF)

    h2 = h_ref[...] + jnp.dot(oc, wo_ref[...],
                              preferred_element_type=_F32) + bo_ref[...]
    xn = _lnorm(h2, g2_ref[...], b2_ref[...]).astype(_BF)
    hh1 = jax.nn.gelu(jnp.dot(xn, w1_ref[...],
                              preferred_element_type=_F32) + b1_ref[...])
    ff = jnp.dot(hh1.astype(_BF), w2_ref[...],
                 preferred_element_type=_F32) + b2b_ref[...]
    return h2 + ff


def _mid_kernel(q_ref, k_ref, v_ref, h_ref, cq_ref, ck_ref, drel_ref,
                wo_ref, bo_ref, g2_ref, b2_ref, w1_ref, b1_ref,
                w2_ref, b2b_ref, g1n_ref, b1n_ref,
                wqn_ref, bqn_ref, wkn_ref, bkn_ref, wvn_ref, bvn_ref,
                ho_ref, qo_ref, ko_ref, vo_ref):
    qi = pl.program_id(0)
    hn = _attn_ffn(qi, q_ref, k_ref, v_ref, h_ref, cq_ref, ck_ref, drel_ref,
                   wo_ref, bo_ref, g2_ref, b2_ref, w1_ref, b1_ref,
                   w2_ref, b2b_ref)
    ho_ref[...] = hn
    xn = _lnorm(hn, g1n_ref[...], b1n_ref[...]).astype(_BF)
    q, k, v = _qkv(xn, wqn_ref, bqn_ref, wkn_ref, bkn_ref, wvn_ref, bvn_ref)
    qo_ref[...] = q
    ko_ref[...] = k
    vo_ref[...] = v


def _last_kernel(q_ref, k_ref, v_ref, h_ref, cq_ref, ck_ref, drel_ref,
                 wo_ref, bo_ref, g2_ref, b2_ref, w1_ref, b1_ref,
                 w2_ref, b2b_ref, gf_ref, bf_ref, hw_ref, hb_ref, o_ref):
    qi = pl.program_id(0)
    hn = _attn_ffn(qi, q_ref, k_ref, v_ref, h_ref, cq_ref, ck_ref, drel_ref,
                   wo_ref, bo_ref, g2_ref, b2_ref, w1_ref, b1_ref,
                   w2_ref, b2b_ref)

    @pl.when(qi == 0)
    def _():
        x0 = hn[0:1, :]
        xn = _lnorm(x0, gf_ref[...], bf_ref[...])
        o_ref[...] = (jnp.dot(xn, hw_ref[...], preferred_element_type=_F32)
                      + hb_ref[...])


def kernel(x, coord, lens, cls_token, fc_w, fc_b, ln1_g, ln1_b, wq, bq,
           wk, bk, wv, bv, wo, bo, rel_bias, ln2_g, ln2_b, w1, b1, w2, b2,
           lnf_g, lnf_b, head_w, head_b):
    xp = jnp.concatenate([
        jnp.zeros((1, _IN_DIM), _F32),
        x[0],
        jnp.zeros((_NPAD - _NP1, _IN_DIM), _F32),
    ], axis=0).astype(_BF)

    pc = jnp.concatenate([
        jnp.zeros((1, 2), _F32),
        coord[0],
        jnp.zeros((_NPAD - _NP1, 2), _F32),
    ], axis=0)
    cq = jnp.pad(pc, ((0, 0), (0, 6))).reshape(_NQT, _TQ, 8)
    ck = jnp.pad(pc.T, ((0, 6), (0, 0)))

    row_spec = pl.BlockSpec((_TQ, _DIM), lambda i: (i, 0))
    vec_spec = pl.BlockSpec((1, _DIM), lambda i: (0, 0))
    full_spec = lambda a, b: pl.BlockSpec((a, b), lambda i: (0, 0))
    slab_spec = pl.BlockSpec((_TQ, _VW), lambda i: (i, 0))

    h, q, k, v = pl.pallas_call(
        _entry_kernel,
        grid=(_NQT,),
        in_specs=[
            pl.BlockSpec((_TQ, _IN_DIM), lambda i: (i, 0)),
            vec_spec,
            full_spec(_IN_DIM, _DIM),
            vec_spec, vec_spec, vec_spec,
            full_spec(_DIM, _DIM), vec_spec,
            full_spec(_DIM, _DIM), vec_spec,
            full_spec(_DIM, _DIM), vec_spec,
        ],
        out_specs=[row_spec, row_spec, row_spec, slab_spec],
        out_shape=[
            jax.ShapeDtypeStruct((_NPAD, _DIM), _F32),
            jax.ShapeDtypeStruct((_NPAD, _DIM), _BF),
            jax.ShapeDtypeStruct((_NPAD, _DIM), _BF),
            jax.ShapeDtypeStruct((_NPAD, _VW), _BF),
        ],
    )(xp, cls_token.reshape(1, _DIM), fc_w.astype(_BF),
      fc_b.reshape(1, _DIM), ln1_g[0].reshape(1, _DIM),
      ln1_b[0].reshape(1, _DIM),
      wq[0].astype(_BF), bq[0].reshape(1, _DIM),
      wk[0].astype(_BF), bk[0].reshape(1, _DIM),
      wv[0].astype(_BF), bv[0].reshape(1, _DIM))

    layer_specs = [
        row_spec,
        pl.BlockSpec((_NPAD, _DIM), lambda i: (0, 0)),
        pl.BlockSpec((_NPAD, _VW), lambda i: (0, 0)),
        row_spec,
        pl.BlockSpec((1, _TQ, 8), lambda i: (i, 0, 0)),
        full_spec(8, _NPAD),
        full_spec(_HEADS, _KB),
        full_spec(_DIM, _DIM), vec_spec,
        vec_spec, vec_spec,
        full_spec(_DIM, _MLP), pl.BlockSpec((1, _MLP), lambda i: (0, 0)),
        full_spec(_MLP, _DIM), vec_spec,
    ]

    h, q, k, v = pl.pallas_call(
        _mid_kernel,
        grid=(_NQT,),
        in_specs=layer_specs + [
            vec_spec, vec_spec,
            full_spec(_DIM, _DIM), vec_spec,
            full_spec(_DIM, _DIM), vec_spec,
            full_spec(_DIM, _DIM), vec_spec,
        ],
        out_specs=[row_spec, row_spec, row_spec, slab_spec],
        out_shape=[
            jax.ShapeDtypeStruct((_NPAD, _DIM), _F32),
            jax.ShapeDtypeStruct((_NPAD, _DIM), _BF),
            jax.ShapeDtypeStruct((_NPAD, _DIM), _BF),
            jax.ShapeDtypeStruct((_NPAD, _VW), _BF),
        ],
    )(q, k, v, h, cq, ck, rel_bias[0].T,
      wo[0].astype(_BF), bo[0].reshape(1, _DIM),
      ln2_g[0].reshape(1, _DIM), ln2_b[0].reshape(1, _DIM),
      w1[0].astype(_BF), b1[0].reshape(1, _MLP),
      w2[0].astype(_BF), b2[0].reshape(1, _DIM),
      ln1_g[1].reshape(1, _DIM), ln1_b[1].reshape(1, _DIM),
      wq[1].astype(_BF), bq[1].reshape(1, _DIM),
      wk[1].astype(_BF), bk[1].reshape(1, _DIM),
      wv[1].astype(_BF), bv[1].reshape(1, _DIM))

    hw = jnp.pad(head_w, ((0, 0), (0, 128 - _NC)))
    hb = jnp.pad(head_b, (0, 128 - _NC)).reshape(1, 128)
    out = pl.pallas_call(
        _last_kernel,
        grid=(_NQT,),
        in_specs=layer_specs + [
            vec_spec, vec_spec,
            full_spec(_DIM, 128),
            pl.BlockSpec((1, 128), lambda i: (0, 0)),
        ],
        out_specs=pl.BlockSpec((1, 128), lambda i: (0, 0)),
        out_shape=jax.ShapeDtypeStruct((1, 128), _F32),
    )(q, k, v, h, cq, ck, rel_bias[1].T,
      wo[1].astype(_BF), bo[1].reshape(1, _DIM),
      ln2_g[1].reshape(1, _DIM), ln2_b[1].reshape(1, _DIM),
      w1[1].astype(_BF), b1[1].reshape(1, _MLP),
      w2[1].astype(_BF), b2[1].reshape(1, _DIM),
      lnf_g.reshape(1, _DIM), lnf_b.reshape(1, _DIM), hw, hb)

    return out[:, :_NC]
